# baseline (speedup 1.0000x reference)
import functools

import jax
import jax.numpy as jnp
from jax import lax
from jax.experimental import pallas as pl
from jax.experimental.pallas import tpu as pltpu
from jax.experimental.pallas import tpu_sc as plsc

_BINS = 1024
_WIDTH = 128
_NC = 2
_NS = 16
_NW = _NC * _NS
_LANES = 16
_C = 128
_SC_ROWS = 256
_GPS = _SC_ROWS // _C


def _sc_body(xt_hbm, table_hbm, out_hbm, x_v, idx_v, table_sh,
             rows0, rows1, xsem, gsem0, gsem1, ssem0, ssem1):
    rows = (rows0, rows1)
    gsem = (gsem0, gsem1)
    ssem = (ssem0, ssem1)
    seq, n_tot = xt_hbm.shape
    n_per_w = n_tot // _NW
    b_per_w = seq * n_per_w
    n_sc = b_per_w // _SC_ROWS
    spp = n_per_w // _SC_ROWS
    sid = lax.axis_index("s")
    wid = sid * _NC + lax.axis_index("c")
    ncol = wid * n_per_w

    rows_per_sub = _BINS // _NS
    pltpu.sync_copy(table_hbm.at[pl.ds(sid * rows_per_sub, rows_per_sub)],
                    table_sh.at[pl.ds(sid * rows_per_sub, rows_per_sub)])

    @pl.when(sid == 0)
    def _last_row():
        pltpu.sync_copy(table_hbm.at[pl.ds(_BINS, 1)],
                        table_sh.at[pl.ds(_BINS, 1)])

    for s in range(seq):
        pltpu.async_copy(xt_hbm.at[s, pl.ds(ncol, n_per_w)],
                         x_v.at[pl.ds(s * n_per_w, n_per_w)], xsem)
    for s in range(seq):
        pltpu.make_async_copy(xt_hbm.at[s, pl.ds(ncol, n_per_w)],
                              x_v.at[pl.ds(s * n_per_w, n_per_w)],
                              xsem).wait()

    def quantize(i):
        for m in range(_SC_ROWS // _LANES):
            off = i * _SC_ROWS + m * _LANES
            xv = x_v[pl.ds(off, _LANES)]
            q = (xv * float(_BINS)).astype(jnp.int32)
            idx_v[pl.ds(off, _LANES)] = jnp.clip(q, 0, _BINS - 1)

    def out_start(i):
        return (i // spp) * n_tot + ncol + (i % spp) * _SC_ROWS

    def gather_start(i, b):
        for j in range(_GPS):
            off = i * _SC_ROWS + j * _C
            pltpu.async_copy(table_hbm.at[idx_v.at[pl.ds(off, _C)]],
                             rows[b].at[pl.ds(j * _C, _C)], gsem[b])

    def gather_wait(i, b):
        for j in range(_GPS):
            off = i * _SC_ROWS + j * _C
            pltpu.make_async_copy(table_hbm.at[idx_v.at[pl.ds(off, _C)]],
                                  rows[b].at[pl.ds(j * _C, _C)],
                                  gsem[b]).wait()

    def scatter_start(i, b):
        pltpu.async_copy(rows[b], out_hbm.at[pl.ds(out_start(i), _SC_ROWS)],
                         ssem[b])

    def scatter_wait(i, b):
        pltpu.make_async_copy(rows[b],
                              out_hbm.at[pl.ds(out_start(i), _SC_ROWS)],
                              ssem[b]).wait()

    quantize(0)
    plsc.subcore_barrier()
    gather_start(0, 0)

    quantize(1)
    gather_start(1, 1)
    gather_wait(0, 0)
    scatter_start(0, 0)

    @pl.loop(1, n_sc - 1, step=2)
    def _main(g):
        for db in range(2):
            i = g + db
            b = (1 + db) % 2
            scatter_wait(i - 1, b ^ 1)
            quantize(i + 1)
            gather_start(i + 1, b ^ 1)
            gather_wait(i, b)
            scatter_start(i, b)

    i_last = n_sc - 1
    b_last = i_last % 2
    scatter_wait(i_last - 1, b_last ^ 1)
    gather_wait(i_last, b_last)
    scatter_start(i_last, b_last)
    scatter_wait(i_last, b_last)


def kernel(x, embed_table):
    n, s = x.shape
    b = n * s
    n_per_w = n // _NW
    b_per_w = s * n_per_w
    mesh = plsc.VectorSubcoreMesh(core_axis_name="c", subcore_axis_name="s")

    call = functools.partial(
        pl.kernel,
        mesh=mesh,
        out_type=jax.ShapeDtypeStruct((b, _WIDTH), jnp.float32),
        compiler_params=pltpu.CompilerParams(use_tc_tiling_on_sc=True),
        scratch_types=(
            [pltpu.VMEM((b_per_w,), jnp.float32),
             pltpu.VMEM((b_per_w,), jnp.int32),
             pltpu.VMEM_SHARED((_BINS + 1, _WIDTH), jnp.float32)]
            + [pltpu.VMEM((_SC_ROWS, _WIDTH), jnp.float32) for _ in range(2)]
            + [pltpu.SemaphoreType.DMA for _ in range(5)]
        ),
    )(_sc_body)

    out = call(x.T, embed_table)
    return out.reshape(s, n, _WIDTH).transpose(1, 0, 2)


# speedup vs baseline: 2.7789x; 2.7789x over previous
import functools

import jax
import jax.numpy as jnp
from jax import lax
from jax.experimental import pallas as pl
from jax.experimental.pallas import tpu as pltpu
from jax.experimental.pallas import tpu_sc as plsc

_BINS = 1024
_WIDTH = 128
_NC = 2
_NS = 16
_NW = _NC * _NS
_LANES = 16
_C = 128
_SC_ROWS = 256
_GPS = _SC_ROWS // _C
_NBUF = 3


def _sc_body(xt_hbm, table_hbm, out_hbm, table_sh, *bufs):
    xb = bufs[0:3]
    idxb = bufs[3:6]
    rows = bufs[6:9]
    xsem = bufs[9:12]
    gsem = bufs[12:15]
    ssem = bufs[15:18]
    seq, n_tot = xt_hbm.shape
    n_per_w = n_tot // _NW
    b_per_w = seq * n_per_w
    n_sc = b_per_w // _SC_ROWS
    spp = n_per_w // _SC_ROWS
    sid = lax.axis_index("s")
    wid = sid * _NC + lax.axis_index("c")
    ncol = wid * n_per_w

    rows_per_sub = _BINS // _NS
    pltpu.sync_copy(table_hbm.at[pl.ds(sid * rows_per_sub, rows_per_sub)],
                    table_sh.at[pl.ds(sid * rows_per_sub, rows_per_sub)])

    @pl.when(sid == 0)
    def _last_row():
        pltpu.sync_copy(table_hbm.at[pl.ds(_BINS, 1)],
                        table_sh.at[pl.ds(_BINS, 1)])

    def src_slice(i):
        return xt_hbm.at[pl.ds(i // spp, 1),
                         pl.ds(ncol + (i % spp) * _SC_ROWS, _SC_ROWS)]

    def xload_start(i, b):
        pltpu.async_copy(src_slice(i), xb[b], xsem[b])

    def xload_wait(i, b):
        pltpu.make_async_copy(src_slice(i), xb[b], xsem[b]).wait()

    def quantize(b):
        for m in range(_SC_ROWS // _LANES):
            xv = xb[b][0, pl.ds(m * _LANES, _LANES)]
            q = (xv * float(_BINS)).astype(jnp.int32)
            idxb[b][pl.ds(m * _LANES, _LANES)] = jnp.clip(q, 0, _BINS - 1)

    def out_start(i):
        return (i // spp) * n_tot + ncol + (i % spp) * _SC_ROWS

    def gather_start(i, b):
        del i
        for j in range(_GPS):
            pltpu.async_copy(table_sh.at[idxb[b].at[pl.ds(j * _C, _C)]],
                             rows[b].at[pl.ds(j * _C, _C)], gsem[b])

    def gather_wait(i, b):
        del i
        for j in range(_GPS):
            pltpu.make_async_copy(table_sh.at[idxb[b].at[pl.ds(j * _C, _C)]],
                                  rows[b].at[pl.ds(j * _C, _C)],
                                  gsem[b]).wait()

    def scatter_start(i, b):
        pltpu.async_copy(rows[b], out_hbm.at[pl.ds(out_start(i), _SC_ROWS)],
                         ssem[b])

    def scatter_wait(i, b):
        pltpu.make_async_copy(rows[b],
                              out_hbm.at[pl.ds(out_start(i), _SC_ROWS)],
                              ssem[b]).wait()

    xload_start(0, 0)
    xload_start(1, 1)
    xload_wait(0, 0)
    quantize(0)
    plsc.subcore_barrier()
    gather_start(0, 0)

    xload_start(2, 2)
    xload_wait(1, 1)
    quantize(1)
    gather_start(1, 1)
    gather_wait(0, 0)
    scatter_start(0, 0)

    xload_start(3, 0)
    xload_wait(2, 2)
    quantize(2)
    gather_start(2, 2)
    gather_wait(1, 1)
    scatter_start(1, 1)

    @pl.loop(2, n_sc - 2, step=_NBUF)
    def _main(g):
        for db in range(_NBUF):
            i = g + db
            m0 = (2 + db) % _NBUF
            m1 = (2 + db + 1) % _NBUF
            m2 = (2 + db + 2) % _NBUF
            scatter_wait(i - 2, m1)
            xload_start(i + 2, m2)
            xload_wait(i + 1, m1)
            quantize(m1)
            gather_start(i + 1, m1)
            gather_wait(i, m0)
            scatter_start(i, m0)

    i = n_sc - 2
    scatter_wait(i - 2, 0)
    xload_wait(i + 1, 0)
    quantize(0)
    gather_start(i + 1, 0)
    gather_wait(i, 2)
    scatter_start(i, 2)

    i = n_sc - 1
    scatter_wait(i - 2, 1)
    gather_wait(i, 0)
    scatter_start(i, 0)

    scatter_wait(n_sc - 2, 2)
    scatter_wait(n_sc - 1, 0)


def kernel(x, embed_table):
    n, s = x.shape
    b = n * s
    mesh = plsc.VectorSubcoreMesh(core_axis_name="c", subcore_axis_name="s")

    call = functools.partial(
        pl.kernel,
        mesh=mesh,
        out_type=jax.ShapeDtypeStruct((b, _WIDTH), jnp.float32),
        compiler_params=pltpu.CompilerParams(use_tc_tiling_on_sc=True),
        scratch_types=(
            [pltpu.VMEM_SHARED((_BINS + 1, _WIDTH), jnp.float32)]
            + [pltpu.VMEM((1, _SC_ROWS), jnp.float32) for _ in range(3)]
            + [pltpu.VMEM((_SC_ROWS,), jnp.int32) for _ in range(3)]
            + [pltpu.VMEM((_SC_ROWS, _WIDTH), jnp.float32) for _ in range(3)]
            + [pltpu.SemaphoreType.DMA for _ in range(9)]
        ),
    )(_sc_body)

    out = call(x.T, embed_table)
    return out.reshape(s, n, _WIDTH).transpose(1, 0, 2)


# speedup vs baseline: 2.7805x; 1.0006x over previous
import functools

import jax
import jax.numpy as jnp
from jax import lax
from jax.experimental import pallas as pl
from jax.experimental.pallas import tpu as pltpu
from jax.experimental.pallas import tpu_sc as plsc

_BINS = 1024
_WIDTH = 128
_NC = 2
_NS = 16
_NW = _NC * _NS
_LANES = 16
_C = 128
_SC_ROWS = 256
_GPS = _SC_ROWS // _C
_NBUF = 3


def _sc_body(xt_hbm, table_hbm, out_hbm, table_sh, *bufs):
    xb = bufs[0:3]
    idxb = bufs[3:6]
    rows = bufs[6:9]
    xsem = bufs[9:12]
    gsem = bufs[12:15]
    ssem = bufs[15:18]
    seq, n_tot = xt_hbm.shape
    n_per_w = n_tot // _NW
    b_per_w = seq * n_per_w
    n_sc = b_per_w // _SC_ROWS
    spp = n_per_w // _SC_ROWS
    sid = lax.axis_index("s")
    wid = sid * _NC + lax.axis_index("c")
    ncol = wid * n_per_w

    rows_per_sub = _BINS // _NS
    pltpu.sync_copy(table_hbm.at[pl.ds(sid * rows_per_sub, rows_per_sub)],
                    table_sh.at[pl.ds(sid * rows_per_sub, rows_per_sub)])

    @pl.when(sid == 0)
    def _last_row():
        pltpu.sync_copy(table_hbm.at[pl.ds(_BINS, 1)],
                        table_sh.at[pl.ds(_BINS, 1)])

    def src_slice(i):
        return xt_hbm.at[pl.ds(i // spp, 1),
                         pl.ds(ncol + (i % spp) * _SC_ROWS, _SC_ROWS)]

    def xload_start(i, b):
        pltpu.async_copy(src_slice(i), xb[b], xsem[b])

    def xload_wait(i, b):
        pltpu.make_async_copy(src_slice(i), xb[b], xsem[b]).wait()

    def quantize(b):
        for m in range(_SC_ROWS // _LANES):
            xv = xb[b][0, pl.ds(m * _LANES, _LANES)]
            q = (xv * float(_BINS)).astype(jnp.int32)
            idxb[b][pl.ds(m * _LANES, _LANES)] = jnp.clip(q, 0, _BINS - 1)

    def out_start(i):
        return (i // spp) * n_tot + ncol + (i % spp) * _SC_ROWS

    def gather_start(i, b):
        del i
        for j in range(_GPS):
            pltpu.async_copy(table_sh.at[idxb[b].at[pl.ds(j * _C, _C)]],
                             rows[b].at[pl.ds(j * _C, _C)], gsem[b])

    def gather_wait(i, b):
        del i
        for j in range(_GPS):
            pltpu.make_async_copy(table_sh.at[idxb[b].at[pl.ds(j * _C, _C)]],
                                  rows[b].at[pl.ds(j * _C, _C)],
                                  gsem[b]).wait()

    def scatter_start(i, b):
        for j in range(_GPS):
            pltpu.async_copy(
                rows[b].at[pl.ds(j * _C, _C)],
                out_hbm.at[pl.ds(out_start(i) + j * _C, _C)], ssem[b])

    def scatter_wait(i, b):
        for j in range(_GPS):
            pltpu.make_async_copy(
                rows[b].at[pl.ds(j * _C, _C)],
                out_hbm.at[pl.ds(out_start(i) + j * _C, _C)],
                ssem[b]).wait()

    xload_start(0, 0)
    xload_start(1, 1)
    xload_wait(0, 0)
    quantize(0)
    plsc.subcore_barrier()
    gather_start(0, 0)

    xload_start(2, 2)
    xload_wait(1, 1)
    quantize(1)
    gather_start(1, 1)
    gather_wait(0, 0)
    scatter_start(0, 0)

    xload_start(3, 0)
    xload_wait(2, 2)
    quantize(2)
    gather_start(2, 2)
    gather_wait(1, 1)
    scatter_start(1, 1)

    @pl.loop(2, n_sc - 2, step=_NBUF)
    def _main(g):
        for db in range(_NBUF):
            i = g + db
            m0 = (2 + db) % _NBUF
            m1 = (2 + db + 1) % _NBUF
            m2 = (2 + db + 2) % _NBUF
            scatter_wait(i - 2, m1)
            xload_start(i + 2, m2)
            xload_wait(i + 1, m1)
            quantize(m1)
            gather_start(i + 1, m1)
            gather_wait(i, m0)
            scatter_start(i, m0)

    i = n_sc - 2
    scatter_wait(i - 2, 0)
    xload_wait(i + 1, 0)
    quantize(0)
    gather_start(i + 1, 0)
    gather_wait(i, 2)
    scatter_start(i, 2)

    i = n_sc - 1
    scatter_wait(i - 2, 1)
    gather_wait(i, 0)
    scatter_start(i, 0)

    scatter_wait(n_sc - 2, 2)
    scatter_wait(n_sc - 1, 0)


def kernel(x, embed_table):
    n, s = x.shape
    b = n * s
    mesh = plsc.VectorSubcoreMesh(core_axis_name="c", subcore_axis_name="s")

    call = functools.partial(
        pl.kernel,
        mesh=mesh,
        out_type=jax.ShapeDtypeStruct((b, _WIDTH), jnp.float32),
        compiler_params=pltpu.CompilerParams(use_tc_tiling_on_sc=True),
        scratch_types=(
            [pltpu.VMEM_SHARED((_BINS + 1, _WIDTH), jnp.float32)]
            + [pltpu.VMEM((1, _SC_ROWS), jnp.float32) for _ in range(3)]
            + [pltpu.VMEM((_SC_ROWS,), jnp.int32) for _ in range(3)]
            + [pltpu.VMEM((_SC_ROWS, _WIDTH), jnp.float32) for _ in range(3)]
            + [pltpu.SemaphoreType.DMA for _ in range(9)]
        ),
    )(_sc_body)

    out = call(x.T, embed_table)
    return out.reshape(s, n, _WIDTH).transpose(1, 0, 2)
